# baseline (speedup 1.0000x reference)
import functools

import jax
import jax.numpy as jnp
from jax import lax
from jax.experimental import pallas as pl
from jax.experimental.pallas import tpu as pltpu
from jax.experimental.pallas import tpu_sc as plsc

_K = 256
_R = 8


def _gate_body(ic_ref, mask_ref, g_ref):
    c = mask_ref.shape[2]
    m_col = mask_ref[0, 0, :].reshape(c, 1)
    m_row = mask_ref[0, 0, :].reshape(1, c)
    counts = jnp.sum((m_row <= m_col).astype(jnp.float32), axis=1, keepdims=True)
    kth = jnp.min(jnp.where(counts >= _K, m_col, jnp.inf))
    thr = jnp.where(ic_ref[0, 0] > 0, kth, -jnp.inf)
    gated = jnp.where(m_col <= thr, 0.0, m_col)
    g_ref[0] = jnp.broadcast_to(gated, (c, 16))


def _gates(mask, ic, b, c):
    return pl.pallas_call(
        _gate_body,
        grid=(b,),
        in_specs=[
            pl.BlockSpec(memory_space=pltpu.SMEM),
            pl.BlockSpec((1, 1, c), lambda i: (i, 0, 0)),
        ],
        out_specs=pl.BlockSpec((1, c, 16), lambda i: (i, 0, 0)),
        out_shape=jax.ShapeDtypeStruct((b, c, 16), jnp.float32),
    )(ic, mask)


def _sc_mul_body(c, l, x_ref, g_ref, o_ref, gb, buf):
    nc = 2
    b = lax.axis_index("s") * nc + lax.axis_index("c")
    nchunks = c // _R

    def chunk_body(ch, _):
        base = b * c + ch * _R
        pltpu.sync_copy(x_ref.at[pl.ds(base, _R)], buf)
        pltpu.sync_copy(g_ref.at[b, pl.ds(ch * _R, _R)], gb)
        gvecs = [gb[r] for r in range(_R)]

        def col(j, carry):
            sl = pl.ds(j * 16, 16)
            for r in range(_R):
                buf[r, sl] = buf[r, sl] * gvecs[r]
            return carry

        lax.fori_loop(0, l // 16, col, 0)
        pltpu.sync_copy(buf, o_ref.at[pl.ds(base, _R)])
        return _

    lax.fori_loop(0, nchunks, chunk_body, 0)


def kernel(x, x_averaged, inactive_channels):
    b, c, l = x.shape
    mask = x_averaged.reshape(b, 1, c)
    ic = jnp.asarray(inactive_channels, jnp.int32).reshape(1, 1)
    g = _gates(mask, ic, b, c)

    mesh = plsc.VectorSubcoreMesh(core_axis_name="c", subcore_axis_name="s")
    sc_mul = functools.partial(
        pl.kernel,
        out_type=jax.ShapeDtypeStruct((b * c, l), jnp.float32),
        mesh=mesh,
        scratch_types=[
            pltpu.VMEM((_R, 16), jnp.float32),
            pltpu.VMEM((_R, l), jnp.float32),
        ],
    )(functools.partial(_sc_mul_body, c, l))
    out = sc_mul(x.reshape(b * c, l), g).reshape(b, c, l)
    return (out, 0.0)


# speedup vs baseline: 1.1310x; 1.1310x over previous
import functools

import jax
import jax.numpy as jnp
from jax import lax
from jax.experimental import pallas as pl
from jax.experimental.pallas import tpu as pltpu
from jax.experimental.pallas import tpu_sc as plsc

_K = 256
_R = 8
_S = 8


def _gate_body(ic_ref, mask_ref, g_ref):
    c = mask_ref.shape[2]
    m_col = mask_ref[0, 0, :].reshape(c, 1)
    m_row = mask_ref[0, 0, :].reshape(1, c)
    counts = jnp.sum((m_row <= m_col).astype(jnp.float32), axis=1, keepdims=True)
    kth = jnp.min(jnp.where(counts >= _K, m_col, jnp.inf))
    thr = jnp.where(ic_ref[0, 0] > 0, kth, -jnp.inf)
    gated = jnp.where(m_col <= thr, 0.0, m_col)
    g_ref[0] = jnp.broadcast_to(gated, (c, 16))


def _gates16(mask, ic, nb, c):
    return pl.pallas_call(
        _gate_body,
        grid=(nb,),
        in_specs=[
            pl.BlockSpec(memory_space=pltpu.SMEM),
            pl.BlockSpec((1, 1, c), lambda i: (i, 0, 0)),
        ],
        out_specs=pl.BlockSpec((1, c, 16), lambda i: (i, 0, 0)),
        out_shape=jax.ShapeDtypeStruct((nb, c, 16), jnp.float32),
    )(ic, mask)


def _sc_mul_body(rows_per_w, l, x_ref, g_ref, o_ref, gb, buf):
    nc = 2
    w = lax.axis_index("s") * nc + lax.axis_index("c")
    nchunks = rows_per_w // _R

    def chunk_body(ch, carry):
        base = w * rows_per_w + ch * _R
        pltpu.sync_copy(x_ref.at[pl.ds(base, _R)], buf)
        pltpu.sync_copy(g_ref.at[pl.ds(base, _R)], gb)
        gvecs = [gb[r] for r in range(_R)]

        def col(j, inner):
            sl = pl.ds(j * 16, 16)
            for r in range(_R):
                buf[r, sl] = buf[r, sl] * gvecs[r]
            return inner

        lax.fori_loop(0, l // 16, col, 0)
        pltpu.sync_copy(buf, o_ref.at[pl.ds(base, _R)])
        return carry

    lax.fori_loop(0, nchunks, chunk_body, 0)


def _tc_fused_body(ic_ref, mask_ref, x_ref, o_ref):
    c = mask_ref.shape[2]
    m_col = mask_ref[0, 0, :].reshape(c, 1)
    m_row = mask_ref[0, 0, :].reshape(1, c)
    counts = jnp.sum((m_row <= m_col).astype(jnp.float32), axis=1, keepdims=True)
    kth = jnp.min(jnp.where(counts >= _K, m_col, jnp.inf))
    thr = jnp.where(ic_ref[0, 0] > 0, kth, -jnp.inf)
    gated = jnp.where(m_col <= thr, 0.0, m_col)
    o_ref[0] = x_ref[0] * gated


def _tc_fused(x, mask, ic, nb, c, l):
    return pl.pallas_call(
        _tc_fused_body,
        grid=(nb,),
        in_specs=[
            pl.BlockSpec(memory_space=pltpu.SMEM),
            pl.BlockSpec((1, 1, c), lambda i: (i, 0, 0)),
            pl.BlockSpec((1, c, l), lambda i: (i, 0, 0)),
        ],
        out_specs=pl.BlockSpec((1, c, l), lambda i: (i, 0, 0)),
        out_shape=jax.ShapeDtypeStruct((nb, c, l), x.dtype),
    )(ic, mask, x)


def kernel(x, x_averaged, inactive_channels):
    b, c, l = x.shape
    mask = x_averaged.reshape(b, 1, c)
    ic = jnp.asarray(inactive_channels, jnp.int32).reshape(1, 1)

    g16 = _gates16(mask[:_S], ic, _S, c).reshape(_S * c, 16)
    rows_per_w = _S * c // 32
    mesh = plsc.VectorSubcoreMesh(core_axis_name="c", subcore_axis_name="s")
    sc_mul = functools.partial(
        pl.kernel,
        out_type=jax.ShapeDtypeStruct((_S * c, l), jnp.float32),
        mesh=mesh,
        scratch_types=[
            pltpu.VMEM((_R, 16), jnp.float32),
            pltpu.VMEM((_R, l), jnp.float32),
        ],
    )(functools.partial(_sc_mul_body, rows_per_w, l))
    y_sc = sc_mul(x[:_S].reshape(_S * c, l), g16).reshape(_S, c, l)

    y_tc = _tc_fused(x[_S:], mask[_S:], ic, b - _S, c, l)

    out = jnp.concatenate([y_sc, y_tc], axis=0)
    return (out, 0.0)


# speedup vs baseline: 3.4515x; 3.0516x over previous
import jax
import jax.numpy as jnp
from jax.experimental import pallas as pl
from jax.experimental.pallas import tpu as pltpu

_K = 256
_NL = 2


def _fused_body(ic_ref, mask_ref, x_ref, o_ref, g_ref):
    c = mask_ref.shape[2]

    @pl.when(pl.program_id(1) == 0)
    def _compute_gate():
        m_col = mask_ref[0, 0, :].reshape(c, 1)
        m_row = mask_ref[0, 0, :].reshape(1, c)
        counts = jnp.sum((m_row <= m_col).astype(jnp.float32), axis=1,
                         keepdims=True)
        kth = jnp.min(jnp.where(counts >= _K, m_col, jnp.inf))
        thr = jnp.where(ic_ref[0, 0] > 0, kth, -jnp.inf)
        g_ref[:, :] = jnp.where(m_col <= thr, 0.0, m_col)

    o_ref[0] = x_ref[0] * g_ref[:, :]


def kernel(x, x_averaged, inactive_channels):
    b, c, l = x.shape
    mask = x_averaged.reshape(b, 1, c)
    ic = jnp.asarray(inactive_channels, jnp.int32).reshape(1, 1)
    lt = l // _NL

    out = pl.pallas_call(
        _fused_body,
        grid=(b, _NL),
        in_specs=[
            pl.BlockSpec(memory_space=pltpu.SMEM),
            pl.BlockSpec((1, 1, c), lambda i, j: (i, 0, 0)),
            pl.BlockSpec((1, c, lt), lambda i, j: (i, 0, j)),
        ],
        out_specs=pl.BlockSpec((1, c, lt), lambda i, j: (i, 0, j)),
        out_shape=jax.ShapeDtypeStruct((b, c, l), x.dtype),
        scratch_shapes=[pltpu.VMEM((c, 1), jnp.float32)],
    )(ic, mask, x)
    return (out, 0.0)


# speedup vs baseline: 3.5199x; 1.0198x over previous
import jax
import jax.numpy as jnp
from jax.experimental import pallas as pl
from jax.experimental.pallas import tpu as pltpu

_K = 256


def _fused_body(ic_ref, mask_ref, x_ref, o_ref):
    c = mask_ref.shape[2]
    i = pl.program_id(0)
    m_col = mask_ref[0, i, :].reshape(c, 1)
    m_row = mask_ref[0, i, :].reshape(1, c)
    counts = jnp.sum((m_row <= m_col).astype(jnp.float32), axis=1, keepdims=True)
    kth = jnp.min(jnp.where(counts >= _K, m_col, jnp.inf))
    thr = jnp.where(ic_ref[0, 0] > 0, kth, -jnp.inf)
    gated = jnp.where(m_col <= thr, 0.0, m_col)
    o_ref[0] = x_ref[0] * gated


def kernel(x, x_averaged, inactive_channels):
    b, c, l = x.shape
    mask = x_averaged.reshape(1, b, c)
    ic = jnp.asarray(inactive_channels, jnp.int32).reshape(1, 1)

    out = pl.pallas_call(
        _fused_body,
        grid=(b,),
        in_specs=[
            pl.BlockSpec(memory_space=pltpu.SMEM),
            pl.BlockSpec((1, b, c), lambda i: (0, 0, 0)),
            pl.BlockSpec((1, c, l), lambda i: (i, 0, 0)),
        ],
        out_specs=pl.BlockSpec((1, c, l), lambda i: (i, 0, 0)),
        out_shape=jax.ShapeDtypeStruct((b, c, l), x.dtype),
    )(ic, mask, x)
    return (out, 0.0)


# speedup vs baseline: 3.5527x; 1.0093x over previous
import jax
import jax.numpy as jnp
from jax.experimental import pallas as pl
from jax.experimental.pallas import tpu as pltpu

_K = 256


def _fused_body(ic_ref, mask_ref, x_ref, o_ref):
    c = mask_ref.shape[2]
    m_col = mask_ref[0, 0, :].reshape(c, 1)
    m_row = mask_ref[0, 0, :].reshape(1, c)
    counts = jnp.sum((m_row <= m_col).astype(jnp.float32), axis=1, keepdims=True)
    kth = jnp.min(jnp.where(counts >= _K, m_col, jnp.inf))
    thr = jnp.where(ic_ref[0, 0] > 0, kth, -jnp.inf)
    gated = jnp.where(m_col <= thr, 0.0, m_col)
    o_ref[0] = x_ref[0] * gated


def kernel(x, x_averaged, inactive_channels):
    b, c, l = x.shape
    mask = x_averaged.reshape(b, 1, c)
    ic = jnp.asarray(inactive_channels, jnp.int32).reshape(1, 1)

    out = pl.pallas_call(
        _fused_body,
        grid=(b,),
        in_specs=[
            pl.BlockSpec(memory_space=pltpu.SMEM),
            pl.BlockSpec((1, 1, c), lambda i: (i, 0, 0)),
            pl.BlockSpec((1, c, l), lambda i: (i, 0, 0)),
        ],
        out_specs=pl.BlockSpec((1, c, l), lambda i: (i, 0, 0)),
        out_shape=jax.ShapeDtypeStruct((b, c, l), x.dtype),
    )(ic, mask, x)
    return (out, 0.0)
